# baseline (speedup 1.0000x reference)
import functools

import jax
import jax.numpy as jnp
from jax import lax
from jax.experimental import pallas as pl
from jax.experimental.pallas import tpu as pltpu
from jax.experimental.pallas import tpu_sc as plsc

_SLOPE = 0.1


def _leaky(v):
    return jnp.where(v >= 0, v, _SLOPE * v)


def _table_body(x_ref, pos_ref, w_ref, b_ref, wp1_ref, o_ref):
    h = jnp.dot(x_ref[...], w_ref[...], preferred_element_type=jnp.float32)
    h = _leaky(h + b_ref[...])
    q = jnp.dot(pos_ref[...], wp1_ref[...], preferred_element_type=jnp.float32)
    pad = jnp.zeros((h.shape[0], o_ref.shape[1] - 2 * h.shape[1]),
                    jnp.float32)
    o_ref[...] = jnp.concatenate([h, q, pad], axis=1)


def _build_table(x2, pos2, W_in, b_in, Wp1, block_rows):
    BN, C_in = x2.shape
    H = W_in.shape[1]
    grid = (BN // block_rows,)
    return pl.pallas_call(
        _table_body,
        grid=grid,
        in_specs=[
            pl.BlockSpec((block_rows, C_in), lambda i: (i, 0)),
            pl.BlockSpec((block_rows, 3), lambda i: (i, 0)),
            pl.BlockSpec((C_in, H), lambda i: (0, 0)),
            pl.BlockSpec((1, H), lambda i: (0, 0)),
            pl.BlockSpec((3, H), lambda i: (0, 0)),
        ],
        out_specs=pl.BlockSpec((block_rows, 128), lambda i: (i, 0)),
        out_shape=jax.ShapeDtypeStruct((BN, 128), jnp.float32),
    )(x2, pos2, W_in, b_in.reshape(1, H), Wp1)


def _sc_gather(table, gidx_flat, d_out):
    E = gidx_flat.shape[0]
    D = table.shape[1]
    W = 80
    NW = 32
    NBUF = 5
    n_chunks = E // (W * NW)
    assert E % (W * NW) == 0 and (n_chunks - NBUF) % NBUF == 0
    mesh = plsc.VectorSubcoreMesh(core_axis_name="c", subcore_axis_name="s")

    del d_out
    @functools.partial(
        pl.kernel,
        out_type=jax.ShapeDtypeStruct((E, D), jnp.float32),
        mesh=mesh,
        scratch_types=[
            pltpu.VMEM((W * n_chunks,), jnp.int32),
            pltpu.VMEM((NBUF, W, D), jnp.float32),
        ] + [pltpu.SemaphoreType.DMA] * (2 * NBUF),
    )
    def gather_kernel(tbl_hbm, idx_hbm, out_hbm, idx_v, rows_v, *sems):
        gsem = sems[:NBUF]
        ssem = sems[NBUF:]
        wid = lax.axis_index("s") * 2 + lax.axis_index("c")
        base = wid * (W * n_chunks)

        def issue_gather(j, b):
            pltpu.async_copy(
                tbl_hbm.at[idx_v.at[pl.ds(j * W, W)]], rows_v.at[b], gsem[b])

        def wait_gather(b):
            pltpu.make_async_copy(
                tbl_hbm.at[idx_v.at[pl.ds(0, W)]], rows_v.at[b], gsem[b]
            ).wait()

        def issue_store(j, b):
            pltpu.async_copy(
                rows_v.at[b], out_hbm.at[pl.ds(base + j * W, W)], ssem[b])

        def wait_store(b):
            pltpu.make_async_copy(
                rows_v.at[b], out_hbm.at[pl.ds(0, W)], ssem[b]
            ).wait()

        pltpu.sync_copy(idx_hbm.at[pl.ds(base, W * n_chunks)], idx_v)
        for j in range(NBUF - 1):
            issue_gather(j, j)
        wait_gather(0)
        issue_gather(NBUF - 1, NBUF - 1)
        issue_store(0, 0)

        @pl.loop(0, (n_chunks - NBUF) // NBUF)
        def _(jj):
            for u in range(NBUF):
                i = 1 + jj * NBUF + u
                b = (1 + u) % NBUF
                bnext = u
                wait_gather(b)
                wait_store(bnext)
                issue_gather(i + NBUF - 1, bnext)
                issue_store(i, b)

        for u in range(NBUF - 1):
            i = n_chunks - NBUF + 1 + u
            b = i % NBUF
            wait_gather(b)
            issue_store(i, b)
        for b in range(NBUF):
            wait_store(b)

    return gather_kernel(table, gidx_flat)


def _combine_body(g_ref, pos_ref, x_ref, w1_ref, b1_ref, w2_ref, b2_ref,
                  wo_ref, bo_ref, o_ref):
    K = g_ref.shape[0]
    H = w2_ref.shape[0]
    posb = pos_ref[...]
    P = posb.shape[0]
    qc = (jnp.dot(posb, w1_ref[...], preferred_element_type=jnp.float32)
          + b1_ref[...])
    w2 = w2_ref[...]
    b2 = b2_ref[...]
    acc = jnp.zeros((P, H), jnp.float32)
    for k in range(K):
        gk = g_ref[k]
        t = _leaky(qc - gk[:, H:2 * H])
        wk = jnp.dot(t, w2, preferred_element_type=jnp.float32) + b2
        acc = acc + wk * gk[:, :H]
    out = jnp.dot(acc, wo_ref[...], preferred_element_type=jnp.float32)
    o_ref[...] = _leaky(out + bo_ref[...] + x_ref[...])


def _combine(gath3, pos2, x2, Wp1, bp1, Wp2, bp2, W_out, b_out, block_rows):
    K, BN, D = gath3.shape
    H = Wp2.shape[0]
    C_out = W_out.shape[1]
    C_in = x2.shape[1]
    grid = (BN // block_rows,)
    return pl.pallas_call(
        _combine_body,
        grid=grid,
        in_specs=[
            pl.BlockSpec((K, block_rows, D), lambda i: (0, i, 0)),
            pl.BlockSpec((block_rows, 3), lambda i: (i, 0)),
            pl.BlockSpec((block_rows, C_in), lambda i: (i, 0)),
            pl.BlockSpec((3, H), lambda i: (0, 0)),
            pl.BlockSpec((1, H), lambda i: (0, 0)),
            pl.BlockSpec((H, H), lambda i: (0, 0)),
            pl.BlockSpec((1, H), lambda i: (0, 0)),
            pl.BlockSpec((H, C_out), lambda i: (0, 0)),
            pl.BlockSpec((1, C_out), lambda i: (0, 0)),
        ],
        out_specs=pl.BlockSpec((block_rows, C_out), lambda i: (i, 0)),
        out_shape=jax.ShapeDtypeStruct((BN, C_out), jnp.float32),
    )(gath3, pos2, x2, Wp1, bp1.reshape(1, H), Wp2, bp2.reshape(1, H),
      W_out, b_out.reshape(1, C_out))


def kernel(x, pos, neighbor_idx, W_in, b_in, Wp1, bp1, Wp2, bp2, W_out, b_out):
    B, N, C_in = x.shape
    K = neighbor_idx.shape[2]
    H = W_in.shape[1]
    BN = B * N
    E = BN * K

    x2 = x.reshape(BN, C_in)
    pos2 = pos.reshape(BN, 3)

    table = _build_table(x2, pos2, W_in, b_in, Wp1, block_rows=2000)

    offs = (jnp.arange(B, dtype=jnp.int32) * N)[:, None, None]
    gidx2 = jnp.transpose(neighbor_idx + offs, (2, 0, 1)).reshape(K, BN)

    S = 5
    BS = BN // S
    outs = []
    for s in range(S):
        sl = slice(s * BS, (s + 1) * BS)
        gath = _sc_gather(table, gidx2[:, sl].reshape(K * BS),
                          d_out=H).reshape(K, BS, 128)
        outs.append(_combine(gath, pos2[sl], x2[sl], Wp1, bp1, Wp2, bp2,
                             W_out, b_out, block_rows=800))
    out2 = jnp.concatenate(outs, axis=0)
    return out2.reshape(B, N, W_out.shape[1])


# speedup vs baseline: 1.1461x; 1.1461x over previous
import functools

import jax
import jax.numpy as jnp
from jax import lax
from jax.experimental import pallas as pl
from jax.experimental.pallas import tpu as pltpu
from jax.experimental.pallas import tpu_sc as plsc

_SLOPE = 0.1


def _leaky(v):
    return jnp.where(v >= 0, v, _SLOPE * v)


def _table_body(x_ref, pos_ref, w_ref, b_ref, wp1_ref, o_ref):
    h = jnp.dot(x_ref[...], w_ref[...], preferred_element_type=jnp.float32)
    h = _leaky(h + b_ref[...])
    q = jnp.dot(pos_ref[...], wp1_ref[...], preferred_element_type=jnp.float32)
    pad = jnp.zeros((h.shape[0], o_ref.shape[1] - 2 * h.shape[1]),
                    jnp.float32)
    o_ref[...] = jnp.concatenate([h, q, pad], axis=1)


def _build_table(x2, pos2, W_in, b_in, Wp1, block_rows):
    BN, C_in = x2.shape
    H = W_in.shape[1]
    grid = (BN // block_rows,)
    return pl.pallas_call(
        _table_body,
        grid=grid,
        in_specs=[
            pl.BlockSpec((block_rows, C_in), lambda i: (i, 0)),
            pl.BlockSpec((block_rows, 3), lambda i: (i, 0)),
            pl.BlockSpec((C_in, H), lambda i: (0, 0)),
            pl.BlockSpec((1, H), lambda i: (0, 0)),
            pl.BlockSpec((3, H), lambda i: (0, 0)),
        ],
        out_specs=pl.BlockSpec((block_rows, 128), lambda i: (i, 0)),
        out_shape=jax.ShapeDtypeStruct((BN, 128), jnp.float32),
    )(x2, pos2, W_in, b_in.reshape(1, H), Wp1)


def _sc_gather(table, gidx_flat, d_out):
    E = gidx_flat.shape[0]
    D = table.shape[1]
    W = 80
    NW = 32
    NBUF = 5
    n_chunks = E // (W * NW)
    assert E % (W * NW) == 0 and (n_chunks - NBUF) % NBUF == 0
    mesh = plsc.VectorSubcoreMesh(core_axis_name="c", subcore_axis_name="s")

    del d_out
    @functools.partial(
        pl.kernel,
        out_type=jax.ShapeDtypeStruct((E, D), jnp.float32),
        mesh=mesh,
        scratch_types=[
            pltpu.VMEM((W * n_chunks,), jnp.int32),
            pltpu.VMEM((NBUF, W, D), jnp.float32),
        ] + [pltpu.SemaphoreType.DMA] * (2 * NBUF),
    )
    def gather_kernel(tbl_hbm, idx_hbm, out_hbm, idx_v, rows_v, *sems):
        gsem = sems[:NBUF]
        ssem = sems[NBUF:]
        wid = lax.axis_index("s") * 2 + lax.axis_index("c")
        base = wid * (W * n_chunks)

        def issue_gather(j, b):
            pltpu.async_copy(
                tbl_hbm.at[idx_v.at[pl.ds(j * W, W)]], rows_v.at[b], gsem[b])

        def wait_gather(b):
            pltpu.make_async_copy(
                tbl_hbm.at[idx_v.at[pl.ds(0, W)]], rows_v.at[b], gsem[b]
            ).wait()

        def issue_store(j, b):
            pltpu.async_copy(
                rows_v.at[b], out_hbm.at[pl.ds(base + j * W, W)], ssem[b])

        def wait_store(b):
            pltpu.make_async_copy(
                rows_v.at[b], out_hbm.at[pl.ds(0, W)], ssem[b]
            ).wait()

        pltpu.sync_copy(idx_hbm.at[pl.ds(base, W * n_chunks)], idx_v)
        for j in range(NBUF - 1):
            issue_gather(j, j)
        wait_gather(0)
        issue_gather(NBUF - 1, NBUF - 1)
        issue_store(0, 0)

        @pl.loop(0, (n_chunks - NBUF) // NBUF)
        def _(jj):
            for u in range(NBUF):
                i = 1 + jj * NBUF + u
                b = (1 + u) % NBUF
                bnext = u
                wait_gather(b)
                wait_store(bnext)
                issue_gather(i + NBUF - 1, bnext)
                issue_store(i, b)

        for u in range(NBUF - 1):
            i = n_chunks - NBUF + 1 + u
            b = i % NBUF
            wait_gather(b)
            issue_store(i, b)
        for b in range(NBUF):
            wait_store(b)

    return gather_kernel(table, gidx_flat)


def _combine_body(g_ref, pos_ref, x_ref, w1_ref, b1_ref, w2_ref, b2_ref,
                  wo_ref, bo_ref, o_ref):
    K = g_ref.shape[0]
    H = w2_ref.shape[0]
    posb = pos_ref[...]
    P = posb.shape[0]
    qc = (jnp.dot(posb, w1_ref[...], preferred_element_type=jnp.float32)
          + b1_ref[...])
    w2 = w2_ref[...]
    b2 = b2_ref[...]
    acc = jnp.zeros((P, H), jnp.float32)
    for k in range(K):
        gk = g_ref[k]
        t = _leaky(qc - gk[:, H:2 * H])
        wk = jnp.dot(t, w2, preferred_element_type=jnp.float32) + b2
        acc = acc + wk * gk[:, :H]
    out = jnp.dot(acc, wo_ref[...], preferred_element_type=jnp.float32)
    o_ref[...] = _leaky(out + bo_ref[...] + x_ref[...])


def _combine(gath3, pos2, x2, Wp1, bp1, Wp2, bp2, W_out, b_out, block_rows):
    K, BN, D = gath3.shape
    H = Wp2.shape[0]
    C_out = W_out.shape[1]
    C_in = x2.shape[1]
    grid = (BN // block_rows,)
    return pl.pallas_call(
        _combine_body,
        grid=grid,
        in_specs=[
            pl.BlockSpec((K, block_rows, D), lambda i: (0, i, 0)),
            pl.BlockSpec((block_rows, 3), lambda i: (i, 0)),
            pl.BlockSpec((block_rows, C_in), lambda i: (i, 0)),
            pl.BlockSpec((3, H), lambda i: (0, 0)),
            pl.BlockSpec((1, H), lambda i: (0, 0)),
            pl.BlockSpec((H, H), lambda i: (0, 0)),
            pl.BlockSpec((1, H), lambda i: (0, 0)),
            pl.BlockSpec((H, C_out), lambda i: (0, 0)),
            pl.BlockSpec((1, C_out), lambda i: (0, 0)),
        ],
        out_specs=pl.BlockSpec((block_rows, C_out), lambda i: (i, 0)),
        out_shape=jax.ShapeDtypeStruct((BN, C_out), jnp.float32),
    )(gath3, pos2, x2, Wp1, bp1.reshape(1, H), Wp2, bp2.reshape(1, H),
      W_out, b_out.reshape(1, C_out))


def kernel(x, pos, neighbor_idx, W_in, b_in, Wp1, bp1, Wp2, bp2, W_out, b_out):
    B, N, C_in = x.shape
    K = neighbor_idx.shape[2]
    H = W_in.shape[1]
    BN = B * N
    E = BN * K

    x2 = x.reshape(BN, C_in)
    pos2 = pos.reshape(BN, 3)

    table = _build_table(x2, pos2, W_in, b_in, Wp1, block_rows=2000)

    offs = (jnp.arange(B, dtype=jnp.int32) * N)[:, None, None]
    gidx2 = jnp.transpose(neighbor_idx + offs, (2, 0, 1)).reshape(K, BN)

    gath = _sc_gather(table, gidx2.reshape(E), d_out=H).reshape(K, BN, 128)
    out2 = _combine(gath, pos2, x2, Wp1, bp1, Wp2, bp2, W_out, b_out,
                    block_rows=2000)
    return out2.reshape(B, N, W_out.shape[1])


# speedup vs baseline: 1.2800x; 1.1169x over previous
import functools

import jax
import jax.numpy as jnp
from jax import lax
from jax.experimental import pallas as pl
from jax.experimental.pallas import tpu as pltpu
from jax.experimental.pallas import tpu_sc as plsc

_SLOPE = 0.1


def _leaky(v):
    return jnp.where(v >= 0, v, _SLOPE * v)


def _pack_bf16_pair(hi_f32, lo_f32):
    hi = lax.bitcast_convert_type(
        hi_f32.astype(jnp.bfloat16).astype(jnp.float32), jnp.uint32)
    lo = lax.bitcast_convert_type(
        lo_f32.astype(jnp.bfloat16).astype(jnp.float32), jnp.uint32)
    return lax.bitcast_convert_type(hi | (lo >> 16), jnp.int32)


def _unpack_hi(words_i32):
    return lax.bitcast_convert_type(
        words_i32 & jnp.int32(-65536), jnp.float32)


def _unpack_lo(words_i32):
    return lax.bitcast_convert_type(
        jnp.left_shift(words_i32, 16), jnp.float32)


def _table_body(x_ref, pos_ref, w_ref, b_ref, wp1_ref, o_ref):
    h = jnp.dot(x_ref[...], w_ref[...], preferred_element_type=jnp.float32)
    h = _leaky(h + b_ref[...])
    q = jnp.dot(pos_ref[...], wp1_ref[...], preferred_element_type=jnp.float32)
    packed = _pack_bf16_pair(q, h)
    pad = jnp.zeros((packed.shape[0], o_ref.shape[1] - packed.shape[1]),
                    jnp.int32)
    o_ref[...] = jnp.concatenate([packed, pad], axis=1)


def _build_table(x2, pos2, W_in, b_in, Wp1, block_rows):
    BN, C_in = x2.shape
    H = W_in.shape[1]
    grid = (BN // block_rows,)
    return pl.pallas_call(
        _table_body,
        grid=grid,
        in_specs=[
            pl.BlockSpec((block_rows, C_in), lambda i: (i, 0)),
            pl.BlockSpec((block_rows, 3), lambda i: (i, 0)),
            pl.BlockSpec((C_in, H), lambda i: (0, 0)),
            pl.BlockSpec((1, H), lambda i: (0, 0)),
            pl.BlockSpec((3, H), lambda i: (0, 0)),
        ],
        out_specs=pl.BlockSpec((block_rows, 128), lambda i: (i, 0)),
        out_shape=jax.ShapeDtypeStruct((BN, 128), jnp.int32),
    )(x2, pos2, W_in, b_in.reshape(1, H), Wp1)


def _sc_gather_pack(table, gidx_flat, K, H, E_real):
    E = gidx_flat.shape[0]
    D = table.shape[1]
    W = 128
    NW = 32
    NBUF = 3
    PPC = W // K
    DP = K * H
    CH = E_real // W
    n_lo = CH // NW
    rem = CH % NW
    assert (n_lo - 2 * NBUF) % NBUF == 0
    mesh = plsc.VectorSubcoreMesh(core_axis_name="c", subcore_axis_name="s")

    @functools.partial(
        pl.kernel,
        out_type=jax.ShapeDtypeStruct((E_real // K, DP), jnp.int32),
        mesh=mesh,
        scratch_types=[
            pltpu.VMEM(((n_lo + 1) * W,), jnp.int32),
            pltpu.VMEM((NBUF, W, D), jnp.int32),
            pltpu.VMEM((NBUF, PPC, DP), jnp.int32),
        ] + [pltpu.SemaphoreType.DMA] * (2 * NBUF),
    )
    def gather_kernel(tbl_hbm, idx_hbm, out_hbm, idx_v, rows_v, pk_v, *sems):
        gsem = sems[:NBUF]
        ssem = sems[NBUF:]
        wid = lax.axis_index("s") * 2 + lax.axis_index("c")
        base_chunk = wid * n_lo + jnp.minimum(wid, rem)
        base_edge = base_chunk * W
        base_pt = base_chunk * PPC
        extra = wid < rem

        def issue_gather(j, b):
            pltpu.async_copy(
                tbl_hbm.at[idx_v.at[pl.ds(j * W, W)]], rows_v.at[b], gsem[b])

        def wait_gather(b):
            pltpu.make_async_copy(
                tbl_hbm.at[idx_v.at[pl.ds(0, W)]], rows_v.at[b], gsem[b]
            ).wait()

        def issue_store(j, b):
            pltpu.async_copy(
                pk_v.at[b], out_hbm.at[pl.ds(base_pt + j * PPC, PPC)], ssem[b])

        def wait_store(b):
            pltpu.make_async_copy(
                pk_v.at[b], out_hbm.at[pl.ds(0, PPC)], ssem[b]
            ).wait()

        def repack(b):
            @pl.loop(0, PPC)
            def _(p):
                for m in range(K):
                    for j2 in range(H // 16):
                        pk_v[b, p, pl.ds(m * H + j2 * 16, 16)] = (
                            rows_v[b, p * K + m, pl.ds(j2 * 16, 16)])

        pltpu.sync_copy(idx_hbm.at[pl.ds(base_edge, (n_lo + 1) * W)], idx_v)
        for j in range(NBUF):
            issue_gather(j, j)
        for u in range(NBUF):
            wait_gather(u)
            repack(u)
            issue_store(u, u)
            issue_gather(u + NBUF, u)

        @pl.loop(0, (n_lo - 2 * NBUF) // NBUF)
        def _(jj):
            for u in range(NBUF):
                i = NBUF + jj * NBUF + u
                wait_gather(u)
                wait_store(u)
                repack(u)
                issue_store(i, u)
                issue_gather(i + NBUF, u)

        for u in range(NBUF):
            i = n_lo - NBUF + u
            wait_gather(u)
            wait_store(u)
            repack(u)
            issue_store(i, u)
            if u == 0:
                @pl.when(extra)
                def _():
                    issue_gather(n_lo, n_lo % NBUF)

        @pl.when(extra)
        def _():
            b = n_lo % NBUF
            wait_gather(b)
            wait_store(b)
            repack(b)
            issue_store(n_lo, b)
        for b in range(NBUF):
            wait_store(b)

    return gather_kernel(table, gidx_flat)


def _combine_body(g_ref, pos_ref, x_ref, w1_ref, b1_ref, w2_ref, b2_ref,
                  wo_ref, bo_ref, o_ref):
    H = w2_ref.shape[0]
    K = g_ref.shape[1] // H
    posb = pos_ref[...]
    P = posb.shape[0]
    qc = (jnp.dot(posb, w1_ref[...], preferred_element_type=jnp.float32)
          + b1_ref[...])
    w2 = w2_ref[...]
    b2 = b2_ref[...]
    acc = jnp.zeros((P, H), jnp.float32)
    for m in range(K):
        wm = g_ref[:, m * H:(m + 1) * H]
        t = _leaky(qc - _unpack_hi(wm))
        wk = jnp.dot(t, w2, preferred_element_type=jnp.float32) + b2
        acc = acc + wk * _unpack_lo(wm)
    out = jnp.dot(acc, wo_ref[...], preferred_element_type=jnp.float32)
    o_ref[...] = _leaky(out + bo_ref[...] + x_ref[...])


def _combine(g2, pos2, x2, Wp1, bp1, Wp2, bp2, W_out, b_out, block_rows):
    BN, DP = g2.shape
    H = Wp2.shape[0]
    C_out = W_out.shape[1]
    C_in = x2.shape[1]
    grid = (BN // block_rows,)
    return pl.pallas_call(
        _combine_body,
        grid=grid,
        in_specs=[
            pl.BlockSpec((block_rows, DP), lambda i: (i, 0)),
            pl.BlockSpec((block_rows, 3), lambda i: (i, 0)),
            pl.BlockSpec((block_rows, C_in), lambda i: (i, 0)),
            pl.BlockSpec((3, H), lambda i: (0, 0)),
            pl.BlockSpec((1, H), lambda i: (0, 0)),
            pl.BlockSpec((H, H), lambda i: (0, 0)),
            pl.BlockSpec((1, H), lambda i: (0, 0)),
            pl.BlockSpec((H, C_out), lambda i: (0, 0)),
            pl.BlockSpec((1, C_out), lambda i: (0, 0)),
        ],
        out_specs=pl.BlockSpec((block_rows, C_out), lambda i: (i, 0)),
        out_shape=jax.ShapeDtypeStruct((BN, C_out), jnp.float32),
    )(g2, pos2, x2, Wp1, bp1.reshape(1, H), Wp2, bp2.reshape(1, H),
      W_out, b_out.reshape(1, C_out))


def kernel(x, pos, neighbor_idx, W_in, b_in, Wp1, bp1, Wp2, bp2, W_out, b_out):
    B, N, C_in = x.shape
    K = neighbor_idx.shape[2]
    H = W_in.shape[1]
    BN = B * N
    E = BN * K

    x2 = x.reshape(BN, C_in)
    pos2 = pos.reshape(BN, 3)

    table = _build_table(x2, pos2, W_in, b_in, Wp1, block_rows=2000)

    offs = (jnp.arange(B, dtype=jnp.int32) * N)[:, None, None]
    gidx = (neighbor_idx + offs).reshape(E)
    W, NW = 128, 32
    e_pad = NW * (E // W // NW + 1) * W
    gidx = jnp.pad(gidx, (0, e_pad - E))

    g2 = _sc_gather_pack(table, gidx, K, H, E_real=E)

    out2 = _combine(g2, pos2, x2, Wp1, bp1, Wp2, bp2, W_out, b_out,
                    block_rows=2000)
    return out2.reshape(B, N, W_out.shape[1])
